# Initial kernel scaffold; baseline (speedup 1.0000x reference)
import jax
import jax.numpy as jnp
from jax.experimental import pallas as pl


def kernel(X_domains, A_predicates, constant_table, W_atom, b_atom):
    raise NotImplementedError("write your pallas kernel here")


# speedup vs baseline: 1.3885x; 1.3885x over previous
import functools

import jax
import jax.numpy as jnp
from jax import lax
from jax.experimental import pallas as pl
from jax.experimental.pallas import tpu as pltpu
from jax.experimental.pallas import tpu_sc as plsc

NC, NS, L = 2, 16, 16
NW = NC * NS
D = 32
CHUNK = 128


def _worker_id():
    return lax.axis_index("s") * NC + lax.axis_index("c")


def _sc_gather(table, x2):
    n_rows = x2.shape[0]
    kc = n_rows // NW
    mesh = plsc.VectorSubcoreMesh(core_axis_name="c", subcore_axis_name="s")

    @functools.partial(
        pl.kernel,
        out_type=jax.ShapeDtypeStruct((n_rows * CHUNK, D), jnp.float32),
        mesh=mesh,
        scratch_types=[
            pltpu.VMEM((kc, CHUNK), jnp.int32),
            pltpu.VMEM((CHUNK, D), jnp.float32),
            pltpu.SemaphoreType.DMA,
        ],
        compiler_params=pltpu.CompilerParams(use_tc_tiling_on_sc=False),
    )
    def k(table_hbm, x_hbm, out_hbm, idx_v, rows_v, sem):
        wid = _worker_id()
        pltpu.sync_copy(x_hbm.at[pl.ds(wid * kc, kc)], idx_v)
        for j in range(kc):
            pltpu.async_copy(table_hbm.at[idx_v.at[j]], rows_v, sem).wait()
            pltpu.sync_copy(rows_v, out_hbm.at[pl.ds((wid * kc + j) * CHUNK, CHUNK)])

    return k(table, x2)


def _tc_project(emb, W, b2):
    n = emb.shape[0]

    def body(e_ref, w_ref, b_ref, p0_ref, p1_ref):
        e = e_ref[...]
        w = w_ref[...]
        p0_ref[...] = jnp.dot(e, w[:D, :], preferred_element_type=jnp.float32) + b_ref[...]
        p1_ref[...] = jnp.dot(e, w[D:, :], preferred_element_type=jnp.float32)

    return pl.pallas_call(
        body,
        out_shape=(
            jax.ShapeDtypeStruct((n, D), jnp.float32),
            jax.ShapeDtypeStruct((n, D), jnp.float32),
        ),
    )(emb, W, b2)


def _sc_atoms(p0, p1, a0r, a1r):
    n_rows = a0r.shape[0]
    kc = n_rows // NW
    mesh = plsc.VectorSubcoreMesh(core_axis_name="c", subcore_axis_name="s")

    @functools.partial(
        pl.kernel,
        out_type=jax.ShapeDtypeStruct((n_rows * CHUNK, D), jnp.float32),
        mesh=mesh,
        scratch_types=[
            pltpu.VMEM((kc, CHUNK), jnp.int32),
            pltpu.VMEM((kc, CHUNK), jnp.int32),
            pltpu.VMEM((CHUNK, D), jnp.float32),
            pltpu.VMEM((CHUNK, D), jnp.float32),
            pltpu.VMEM((CHUNK, D), jnp.float32),
            pltpu.SemaphoreType.DMA,
            pltpu.SemaphoreType.DMA,
        ],
        compiler_params=pltpu.CompilerParams(use_tc_tiling_on_sc=False),
    )
    def k(p0_hbm, p1_hbm, a0_hbm, a1_hbm, out_hbm,
          idx0, idx1, r0, r1, ob, sem0, sem1):
        wid = _worker_id()
        pltpu.sync_copy(a0_hbm.at[pl.ds(wid * kc, kc)], idx0)
        pltpu.sync_copy(a1_hbm.at[pl.ds(wid * kc, kc)], idx1)
        for j in range(kc):
            cp0 = pltpu.async_copy(p0_hbm.at[idx0.at[j]], r0, sem0)
            cp1 = pltpu.async_copy(p1_hbm.at[idx1.at[j]], r1, sem1)
            cp0.wait()
            cp1.wait()

            def row(rr, _):
                for h in range(D // L):
                    s = r0[rr, pl.ds(h * L, L)] + r1[rr, pl.ds(h * L, L)]
                    t = jnp.exp(-2.0 * jnp.abs(s))
                    y = (1.0 - t) / (1.0 + t)
                    ob[rr, pl.ds(h * L, L)] = jnp.where(s < 0, -y, y)
                return 0

            lax.fori_loop(0, CHUNK, row, 0)
            pltpu.sync_copy(ob, out_hbm.at[pl.ds((wid * kc + j) * CHUNK, CHUNK)])

    return k(p0, p1, a0r, a1r)


def kernel(X_domains, A_predicates, constant_table, W_atom, b_atom):
    n_const = X_domains.shape[0]
    n_atoms = A_predicates.shape[0]
    x2 = X_domains.reshape(n_const // CHUNK, CHUNK)
    emb = _sc_gather(constant_table, x2)
    p0, p1 = _tc_project(emb, W_atom, b_atom.reshape(1, D))
    a0 = A_predicates[:, 0].reshape(n_atoms // CHUNK, CHUNK)
    a1 = A_predicates[:, 1].reshape(n_atoms // CHUNK, CHUNK)
    return _sc_atoms(p0, p1, a0, a1)


# speedup vs baseline: 2.0034x; 1.4429x over previous
import functools

import jax
import jax.numpy as jnp
from jax import lax
from jax.experimental import pallas as pl
from jax.experimental.pallas import tpu as pltpu
from jax.experimental.pallas import tpu_sc as plsc

NC, NS, L = 2, 16, 16
NW = NC * NS
D = 32
CHUNK = 128
BC = 64


def _worker_id():
    return lax.axis_index("s") * NC + lax.axis_index("c")


def _sc_gather_tiled(table, x_flat):
    n = x_flat.shape[0]
    per_w = n // NW
    K = 16
    nch = per_w // K
    mesh = plsc.VectorSubcoreMesh(core_axis_name="c", subcore_axis_name="s")

    @functools.partial(
        pl.kernel,
        out_type=jax.ShapeDtypeStruct((n, D), jnp.float32),
        mesh=mesh,
        scratch_types=[
            pltpu.VMEM((per_w,), jnp.int32),
            pltpu.VMEM((K, D), jnp.float32),
            pltpu.SemaphoreType.DMA,
        ],
        compiler_params=pltpu.CompilerParams(needs_layout_passes=False),
    )
    def k(t_hbm, x_hbm, out_hbm, xw, rows, sem):
        wid = _worker_id()
        pltpu.sync_copy(x_hbm.at[pl.ds(wid * per_w, per_w)], xw)

        def chunk(c, _):
            xv = xw[pl.ds(c * K, K)]
            cps = []
            for i in range(K):
                xi = xv[i]
                cps.append(pltpu.async_copy(
                    t_hbm.at[pl.ds(xi, 1)], rows.at[pl.ds(i, 1)], sem))
            for cp in cps:
                cp.wait()
            pltpu.sync_copy(rows, out_hbm.at[pl.ds(wid * per_w + c * K, K)])
            return 0

        lax.fori_loop(0, nch, chunk, 0)

    return k(table, x_flat)


def _tc_project(emb, W, b2):
    n = emb.shape[0]

    def body(e_ref, w_ref, b_ref, p0_ref, p1_ref):
        e = e_ref[...]
        w = w_ref[...]
        p0_ref[...] = jnp.dot(e, w[:D, :], preferred_element_type=jnp.float32) + b_ref[...]
        p1_ref[...] = jnp.dot(e, w[D:, :], preferred_element_type=jnp.float32)

    return pl.pallas_call(
        body,
        out_shape=(
            jax.ShapeDtypeStruct((n, D), jnp.float32),
            jax.ShapeDtypeStruct((n, D), jnp.float32),
        ),
    )(emb, W, b2)


def _sc_atoms(p0, p1, a0r, a1r):
    n_rows = a0r.shape[0]
    kc = n_rows // NW
    mesh = plsc.VectorSubcoreMesh(core_axis_name="c", subcore_axis_name="s")

    @functools.partial(
        pl.kernel,
        out_type=jax.ShapeDtypeStruct((n_rows * CHUNK, D), jnp.float32),
        mesh=mesh,
        scratch_types=[
            pltpu.VMEM((kc, CHUNK), jnp.int32),
            pltpu.VMEM((kc, CHUNK), jnp.int32),
            pltpu.VMEM((CHUNK, D), jnp.float32),
            pltpu.VMEM((CHUNK, D), jnp.float32),
            pltpu.VMEM((CHUNK, D), jnp.float32),
            pltpu.SemaphoreType.DMA,
            pltpu.SemaphoreType.DMA,
        ],
        compiler_params=pltpu.CompilerParams(use_tc_tiling_on_sc=False),
    )
    def k(p0_hbm, p1_hbm, a0_hbm, a1_hbm, out_hbm,
          idx0, idx1, r0, r1, ob, sem0, sem1):
        wid = _worker_id()
        pltpu.sync_copy(a0_hbm.at[pl.ds(wid * kc, kc)], idx0)
        pltpu.sync_copy(a1_hbm.at[pl.ds(wid * kc, kc)], idx1)
        for j in range(kc):
            cp0 = pltpu.async_copy(p0_hbm.at[idx0.at[j]], r0, sem0)
            cp1 = pltpu.async_copy(p1_hbm.at[idx1.at[j]], r1, sem1)
            cp0.wait()
            cp1.wait()

            def row(rr, _):
                for h in range(D // L):
                    s = r0[rr, pl.ds(h * L, L)] + r1[rr, pl.ds(h * L, L)]
                    t = jnp.exp(-2.0 * jnp.abs(s))
                    y = (1.0 - t) / (1.0 + t)
                    ob[rr, pl.ds(h * L, L)] = jnp.where(s < 0, -y, y)
                return 0

            lax.fori_loop(0, CHUNK, row, 0)
            pltpu.sync_copy(ob, out_hbm.at[pl.ds((wid * kc + j) * CHUNK, CHUNK)])

    return k(p0, p1, a0r, a1r)


def kernel(X_domains, A_predicates, constant_table, W_atom, b_atom):
    n_const = X_domains.shape[0]
    n_atoms = A_predicates.shape[0]
    emb = _sc_gather_tiled(constant_table, X_domains)
    p0, p1 = _tc_project(emb, W_atom, b_atom.reshape(1, D))
    a0 = A_predicates[:, 0].reshape(n_atoms // CHUNK, CHUNK)
    a1 = A_predicates[:, 1].reshape(n_atoms // CHUNK, CHUNK)
    return _sc_atoms(p0, p1, a0, a1)


# speedup vs baseline: 2.0034x; 1.0000x over previous
import functools

import jax
import jax.numpy as jnp
from jax import lax
from jax.experimental import pallas as pl
from jax.experimental.pallas import tpu as pltpu
from jax.experimental.pallas import tpu_sc as plsc

NC, NS, L = 2, 16, 16
NW = NC * NS
D = 32
CHUNK = 128
BC = 64


def _worker_id():
    return lax.axis_index("s") * NC + lax.axis_index("c")


def _sc_gather_tiled(table, x_flat):
    n = x_flat.shape[0]
    per_w = n // NW
    K = 16
    nch = per_w // K
    mesh = plsc.VectorSubcoreMesh(core_axis_name="c", subcore_axis_name="s")

    @functools.partial(
        pl.kernel,
        out_type=jax.ShapeDtypeStruct((n, D), jnp.float32),
        mesh=mesh,
        scratch_types=[
            pltpu.VMEM((per_w,), jnp.int32),
            pltpu.VMEM((K, D), jnp.float32),
            pltpu.SemaphoreType.DMA,
        ],
    )
    def k(t_hbm, x_hbm, out_hbm, xw, rows, sem):
        wid = _worker_id()
        pltpu.sync_copy(x_hbm.at[pl.ds(wid * per_w, per_w)], xw)

        def chunk(c, _):
            xv = xw[pl.ds(c * K, K)]
            cps = []
            for i in range(K):
                xi = xv[i]
                cps.append(pltpu.async_copy(
                    t_hbm.at[pl.ds(xi, 1)], rows.at[pl.ds(i, 1)], sem))
            for cp in cps:
                cp.wait()
            pltpu.sync_copy(rows, out_hbm.at[pl.ds(wid * per_w + c * K, K)])
            return 0

        lax.fori_loop(0, nch, chunk, 0)

    return k(table, x_flat)


def _tc_project(emb, W, b2):
    n = emb.shape[0]

    def body(e_ref, w_ref, b_ref, p0_ref, p1_ref):
        e = e_ref[...]
        w = w_ref[...]
        p0_ref[...] = jnp.dot(e, w[:D, :], preferred_element_type=jnp.float32) + b_ref[...]
        p1_ref[...] = jnp.dot(e, w[D:, :], preferred_element_type=jnp.float32)

    return pl.pallas_call(
        body,
        out_shape=(
            jax.ShapeDtypeStruct((n, D), jnp.float32),
            jax.ShapeDtypeStruct((n, D), jnp.float32),
        ),
    )(emb, W, b2)


def _sc_atoms(p0, p1, a0r, a1r):
    n_rows = a0r.shape[0]
    kc = n_rows // NW
    mesh = plsc.VectorSubcoreMesh(core_axis_name="c", subcore_axis_name="s")

    @functools.partial(
        pl.kernel,
        out_type=jax.ShapeDtypeStruct((n_rows * CHUNK, D), jnp.float32),
        mesh=mesh,
        scratch_types=[
            pltpu.VMEM((kc, CHUNK), jnp.int32),
            pltpu.VMEM((kc, CHUNK), jnp.int32),
            pltpu.VMEM((CHUNK, D), jnp.float32),
            pltpu.VMEM((CHUNK, D), jnp.float32),
            pltpu.VMEM((CHUNK, D), jnp.float32),
            pltpu.SemaphoreType.DMA,
            pltpu.SemaphoreType.DMA,
        ],
        compiler_params=pltpu.CompilerParams(use_tc_tiling_on_sc=False),
    )
    def k(p0_hbm, p1_hbm, a0_hbm, a1_hbm, out_hbm,
          idx0, idx1, r0, r1, ob, sem0, sem1):
        wid = _worker_id()
        pltpu.sync_copy(a0_hbm.at[pl.ds(wid * kc, kc)], idx0)
        pltpu.sync_copy(a1_hbm.at[pl.ds(wid * kc, kc)], idx1)
        for j in range(kc):
            cp0 = pltpu.async_copy(p0_hbm.at[idx0.at[j]], r0, sem0)
            cp1 = pltpu.async_copy(p1_hbm.at[idx1.at[j]], r1, sem1)
            cp0.wait()
            cp1.wait()

            def row(rr, _):
                for h in range(D // L):
                    s = r0[rr, pl.ds(h * L, L)] + r1[rr, pl.ds(h * L, L)]
                    t = jnp.exp(-2.0 * jnp.abs(s))
                    y = (1.0 - t) / (1.0 + t)
                    ob[rr, pl.ds(h * L, L)] = jnp.where(s < 0, -y, y)
                return 0

            lax.fori_loop(0, CHUNK, row, 0)
            pltpu.sync_copy(ob, out_hbm.at[pl.ds((wid * kc + j) * CHUNK, CHUNK)])

    return k(p0, p1, a0r, a1r)


def kernel(X_domains, A_predicates, constant_table, W_atom, b_atom):
    n_const = X_domains.shape[0]
    n_atoms = A_predicates.shape[0]
    emb = _sc_gather_tiled(constant_table, X_domains)
    p0, p1 = _tc_project(emb, W_atom, b_atom.reshape(1, D))
    a0 = A_predicates[:, 0].reshape(n_atoms // CHUNK, CHUNK)
    a1 = A_predicates[:, 1].reshape(n_atoms // CHUNK, CHUNK)
    return _sc_atoms(p0, p1, a0, a1)
